# baseline (speedup 1.0000x reference)
import functools
import jax
import jax.numpy as jnp
from jax import lax
from jax.experimental import pallas as pl
from jax.experimental.pallas import tpu as pltpu
from jax.experimental.pallas import tpu_sc as plsc

_NC, _NS = 2, 16
_NW = _NC * _NS
_R = 8
_Q = 4


def _sc_body(w_hbm, out_hbm, stage, sems):
    B, ROW = out_hbm.shape
    per_w = B // _NW
    n_chunks = per_w // _R
    c = lax.axis_index("c")
    s = lax.axis_index("s")
    wid = s * _NC + c
    base = wid * per_w
    for r in range(_R):
        pltpu.sync_copy(w_hbm, stage.at[r])

    def loop(i, carry):
        @pl.when(i >= _Q)
        def _():
            pltpu.make_async_copy(
                stage, out_hbm.at[pl.ds(base + (i - _Q) * _R, _R), :], sems.at[i % _Q]
            ).wait()
        pltpu.make_async_copy(
            stage, out_hbm.at[pl.ds(base + i * _R, _R), :], sems.at[i % _Q]
        ).start()
        return carry

    lax.fori_loop(0, n_chunks, loop, 0)
    for q in range(_Q):
        i = n_chunks - _Q + q
        pltpu.make_async_copy(
            stage, out_hbm.at[pl.ds(base + i * _R, _R), :], sems.at[i % _Q]
        ).wait()


def kernel(x, W):
    B, S = x.shape
    M, D = W.shape
    ROW = S * D
    Wf = W[:S].reshape(ROW)
    mesh = plsc.VectorSubcoreMesh(core_axis_name="c", subcore_axis_name="s")
    k = functools.partial(
        pl.kernel,
        mesh=mesh,
        out_type=jax.ShapeDtypeStruct((B, ROW), jnp.float32),
        scratch_types=[
            pltpu.VMEM((_R, ROW), jnp.float32),
            pltpu.SemaphoreType.DMA((_Q,)),
        ],
    )(_sc_body)
    out = k(Wf)
    return out.reshape(B, S, D)


# speedup vs baseline: 1.0054x; 1.0054x over previous
import functools
import jax
import jax.numpy as jnp
from jax import lax
from jax.experimental import pallas as pl
from jax.experimental.pallas import tpu as pltpu
from jax.experimental.pallas import tpu_sc as plsc

_NC, _NS = 2, 16
_NW = _NC * _NS
_R = 8
_Q = 4


def _sc_body(w_hbm, out_hbm, stage, sems):
    B, ROW = out_hbm.shape
    per_w = B // _NW
    n_chunks = per_w // _R
    c = lax.axis_index("c")
    s = lax.axis_index("s")
    wid = s * _NC + c
    base = wid * per_w
    for r in range(_R):
        pltpu.sync_copy(w_hbm, stage.at[r])

    def loop(i, carry):
        pltpu.make_async_copy(
            stage, out_hbm.at[pl.ds(base + i * _R, _R), :], sems.at[0]
        ).start()
        return carry

    lax.fori_loop(0, n_chunks, loop, 0)

    def drain(i, carry):
        pltpu.make_async_copy(
            stage, out_hbm.at[pl.ds(base, _R), :], sems.at[0]
        ).wait()
        return carry

    lax.fori_loop(0, n_chunks, drain, 0)


def kernel(x, W):
    B, S = x.shape
    M, D = W.shape
    ROW = S * D
    Wf = W[:S].reshape(ROW)
    mesh = plsc.VectorSubcoreMesh(core_axis_name="c", subcore_axis_name="s")
    k = functools.partial(
        pl.kernel,
        mesh=mesh,
        out_type=jax.ShapeDtypeStruct((B, ROW), jnp.float32),
        scratch_types=[
            pltpu.VMEM((_R, ROW), jnp.float32),
            pltpu.SemaphoreType.DMA((_Q,)),
        ],
    )(_sc_body)
    out = k(Wf)
    return out.reshape(B, S, D)
